# baseline (speedup 1.0000x reference)
import jax
import jax.numpy as jnp
from jax import lax
from jax.experimental import pallas as pl
from jax.experimental.pallas import tpu as pltpu
from jax.experimental.pallas import tpu_sc as plsc

NUM_CORES = 2
NUM_SUBCORES = 16
NW = NUM_CORES * NUM_SUBCORES

BATCH = 16384
LEVELS = 7
DIM = 64
VOCAB_TOTAL = 508
ROWS = BATCH * LEVELS
RPW = ROWS // NW
CH = 448
BPC = CH // LEVELS
NCH = RPW // CH


def _body(paths_ref, table_ref, out_ref, tbuf, pbufA, pbufB, obufA, obufB,
          lsem, isem, ssem):
    wid = lax.axis_index("s") * NUM_CORES + lax.axis_index("c")
    bw = wid * (BATCH // NW)

    c_t = pltpu.async_copy(table_ref, tbuf, lsem)
    pbufs = [pbufA, pbufB]
    obufs = [obufA, obufB]
    i_copy = {
        0: pltpu.async_copy(paths_ref.at[pl.ds(bw, BPC)], pbufA, isem.at[0])
    }
    c_t.wait()

    iota = lax.iota(jnp.int32, 16)
    sevens = jnp.full((16,), LEVELS, jnp.int32)
    fours = jnp.full((16,), 4, jnp.int32)
    s_copy = {}

    for ch in range(NCH):
        if ch + 1 < NCH:
            i_copy[ch + 1] = pltpu.async_copy(
                paths_ref.at[pl.ds(bw + (ch + 1) * BPC, BPC)],
                pbufs[(ch + 1) % 2], isem.at[(ch + 1) % 2])
        i_copy[ch].wait()
        if ch >= 2:
            s_copy[ch - 2].wait()
        pb = pbufs[ch % 2]
        ob = obufs[ch % 2]

        def fill(i, carry, ch=ch, pb=pb, ob=ob):
            t = ch * CH + i * 16
            brel = lax.div(t + iota, sevens) - ch * BPC
            l16 = lax.rem(t + iota, sevens)
            p = plsc.load_gather(pb, [brel, l16])
            iv = p + lax.shift_left(fours, l16) - 4
            for j in range(16):
                ivj = lax.gather(
                    iv, jnp.full((16, 1), j, jnp.int32),
                    dimension_numbers=lax.GatherDimensionNumbers(
                        offset_dims=(), collapsed_slice_dims=(0,),
                        start_index_map=(0,)),
                    slice_sizes=(1,),
                    mode=lax.GatherScatterMode.PROMISE_IN_BOUNDS)
                for g in range(DIM // 16):
                    v = plsc.load_gather(tbuf, [ivj, g * 16 + iota])
                    ob[i * 16 + j, pl.ds(g * 16, 16)] = v
            return carry

        lax.fori_loop(0, CH // 16, fill, 0)
        s_copy[ch] = pltpu.async_copy(
            ob, out_ref.at[pl.ds(wid * RPW + ch * CH, CH)],
            ssem.at[ch % 2])

    s_copy[NCH - 2].wait()
    s_copy[NCH - 1].wait()


@jax.jit
def kernel(paths, W0, W1, W2, W3, W4, W5, W6):
    table = jnp.concatenate([W0, W1, W2, W3, W4, W5, W6], axis=0)

    mesh = plsc.VectorSubcoreMesh(core_axis_name="c", subcore_axis_name="s")
    out = pl.kernel(
        _body,
        out_type=jax.ShapeDtypeStruct((ROWS, DIM), jnp.float32),
        mesh=mesh,
        compiler_params=pltpu.CompilerParams(
            use_tc_tiling_on_sc=False, needs_layout_passes=False),
        scratch_types=[
            pltpu.VMEM((VOCAB_TOTAL, DIM), jnp.float32),
            pltpu.VMEM((BPC, LEVELS), jnp.int32),
            pltpu.VMEM((BPC, LEVELS), jnp.int32),
            pltpu.VMEM((CH, DIM), jnp.float32),
            pltpu.VMEM((CH, DIM), jnp.float32),
            pltpu.SemaphoreType.DMA,
            pltpu.SemaphoreType.DMA((2,)),
            pltpu.SemaphoreType.DMA((2,)),
        ],
    )(paths, table)
    return out.reshape(BATCH, LEVELS * DIM)


# speedup vs baseline: 1.0145x; 1.0145x over previous
import jax
import jax.numpy as jnp
from jax import lax
from jax.experimental import pallas as pl
from jax.experimental.pallas import tpu as pltpu
from jax.experimental.pallas import tpu_sc as plsc

NUM_CORES = 2
NUM_SUBCORES = 16
NW = NUM_CORES * NUM_SUBCORES

BATCH = 16384
LEVELS = 7
DIM = 64
VOCAB_TOTAL = 508
ROWS = BATCH * LEVELS
RPW = ROWS // NW
BPC = 32
CH = BPC * LEVELS
NCH = RPW // CH


def _body(paths_ref, table_ref, out_ref, tbuf, pbufA, pbufB, obufA, obufB,
          lsem, isem, ssem):
    wid = lax.axis_index("s") * NUM_CORES + lax.axis_index("c")
    bw = pl.multiple_of(wid * (BATCH // NW), 64)
    ow = pl.multiple_of(wid * (RPW // 2), 64)

    c_t = pltpu.async_copy(table_ref, tbuf, lsem)
    pbufs = [pbufA, pbufB]
    obufs = [obufA, obufB]
    i_copy = {
        0: pltpu.async_copy(paths_ref.at[pl.ds(bw, BPC)], pbufA,
                            isem.at[0])
    }
    c_t.wait()

    iota = lax.iota(jnp.int32, 16)
    sevens = jnp.full((16,), LEVELS, jnp.int32)
    fours = jnp.full((16,), 4, jnp.int32)
    s_copy = {}

    for ch in range(NCH):
        if ch + 1 < NCH:
            i_copy[ch + 1] = pltpu.async_copy(
                paths_ref.at[pl.ds(pl.multiple_of(bw + (ch + 1) * BPC, 8),
                                   BPC)],
                pbufs[(ch + 1) % 2], isem.at[(ch + 1) % 2])
        i_copy[ch].wait()
        if ch >= 2:
            s_copy[ch - 2].wait()
        pb = pbufs[ch % 2]
        ob = obufs[ch % 2]

        def fill(i, carry, ch=ch, pb=pb, ob=ob):
            t = ch * CH + i * 16
            brel = lax.div(t + iota, sevens) - ch * BPC
            l16 = lax.rem(t + iota, sevens)
            p = plsc.load_gather(pb, [brel, l16])
            iv = p + lax.shift_left(fours, l16) - 4
            for j in range(16):
                ivj = lax.gather(
                    iv, jnp.full((16, 1), j, jnp.int32),
                    dimension_numbers=lax.GatherDimensionNumbers(
                        offset_dims=(), collapsed_slice_dims=(0,),
                        start_index_map=(0,)),
                    slice_sizes=(1,),
                    mode=lax.GatherScatterMode.PROMISE_IN_BOUNDS)
                for g in range(DIM // 16):
                    v = plsc.load_gather(tbuf, [ivj, g * 16 + iota])
                    ob[i * 8 + j // 2,
                       pl.ds((j % 2) * DIM + g * 16, 16)] = v
            return carry

        lax.fori_loop(0, CH // 16, fill, 0)
        s_copy[ch] = pltpu.async_copy(
            ob, out_ref.at[pl.ds(pl.multiple_of(ow + ch * (CH // 2), 8),
                                 CH // 2)],
            ssem.at[ch % 2])

    s_copy[NCH - 2].wait()
    s_copy[NCH - 1].wait()


@jax.jit
def kernel(paths, W0, W1, W2, W3, W4, W5, W6):
    table = jnp.concatenate([W0, W1, W2, W3, W4, W5, W6], axis=0)

    mesh = plsc.VectorSubcoreMesh(core_axis_name="c", subcore_axis_name="s")
    out = pl.kernel(
        _body,
        out_type=jax.ShapeDtypeStruct((ROWS // 2, 2 * DIM), jnp.float32),
        mesh=mesh,
        compiler_params=pltpu.CompilerParams(
            use_tc_tiling_on_sc=True, needs_layout_passes=False),
        scratch_types=[
            pltpu.VMEM((VOCAB_TOTAL, DIM), jnp.float32),
            pltpu.VMEM((BPC, LEVELS), jnp.int32),
            pltpu.VMEM((BPC, LEVELS), jnp.int32),
            pltpu.VMEM((CH // 2, 2 * DIM), jnp.float32),
            pltpu.VMEM((CH // 2, 2 * DIM), jnp.float32),
            pltpu.SemaphoreType.DMA,
            pltpu.SemaphoreType.DMA((2,)),
            pltpu.SemaphoreType.DMA((2,)),
        ],
    )(paths, table)
    return out.reshape(BATCH, LEVELS * DIM)


# speedup vs baseline: 1.0720x; 1.0566x over previous
import jax
import jax.numpy as jnp
from jax import lax
from jax.experimental import pallas as pl
from jax.experimental.pallas import tpu as pltpu
from jax.experimental.pallas import tpu_sc as plsc

NUM_CORES = 2
NUM_SUBCORES = 16
NW = NUM_CORES * NUM_SUBCORES

BATCH = 16384
LEVELS = 7
DIM = 64
VOCAB_TOTAL = 508
ROWS = BATCH * LEVELS
RPW = ROWS // NW
BPC = 32
CH = BPC * LEVELS
NCH = RPW // CH


def _body(paths_ref, w0, w1, w2, w3, w4, w5, w6, out_ref,
          tbuf, pbufA, pbufB, obufA, obufB, lsem, isem, ssem):
    wid = lax.axis_index("s") * NUM_CORES + lax.axis_index("c")
    bw = pl.multiple_of(wid * (BATCH // NW), 64)
    ow = pl.multiple_of(wid * (RPW // 2), 64)

    t_copies = []
    for l, wref in enumerate([w0, w1, w2, w3, w4, w5, w6]):
        poff = (4 << l) & ~7
        t_copies.append(pltpu.async_copy(
            wref, tbuf.at[pl.ds(poff, 4 << l)], lsem))
    pbufs = [pbufA, pbufB]
    obufs = [obufA, obufB]
    i_copy = {
        0: pltpu.async_copy(paths_ref.at[pl.ds(bw, BPC)], pbufA,
                            isem.at[0])
    }
    for c in t_copies:
        c.wait()

    iota = lax.iota(jnp.int32, 16)
    sevens = jnp.full((16,), LEVELS, jnp.int32)
    fours = jnp.full((16,), 4, jnp.int32)
    s_copy = {}

    for ch in range(NCH):
        if ch + 1 < NCH:
            i_copy[ch + 1] = pltpu.async_copy(
                paths_ref.at[pl.ds(pl.multiple_of(bw + (ch + 1) * BPC, 8),
                                   BPC)],
                pbufs[(ch + 1) % 2], isem.at[(ch + 1) % 2])
        i_copy[ch].wait()
        if ch >= 2:
            s_copy[ch - 2].wait()
        pb = pbufs[ch % 2]
        ob = obufs[ch % 2]

        def fill(i, carry, ch=ch, pb=pb, ob=ob):
            t = ch * CH + i * 16
            brel = lax.div(t + iota, sevens) - ch * BPC
            l16 = lax.rem(t + iota, sevens)
            p = plsc.load_gather(pb, [brel, l16])
            iv = p + jnp.bitwise_and(lax.shift_left(fours, l16),
                                     jnp.full((16,), -8, jnp.int32))
            for j in range(16):
                ivj = lax.gather(
                    iv, jnp.full((16, 1), j, jnp.int32),
                    dimension_numbers=lax.GatherDimensionNumbers(
                        offset_dims=(), collapsed_slice_dims=(0,),
                        start_index_map=(0,)),
                    slice_sizes=(1,),
                    mode=lax.GatherScatterMode.PROMISE_IN_BOUNDS)
                for g in range(DIM // 16):
                    v = plsc.load_gather(tbuf, [ivj, g * 16 + iota])
                    ob[i * 8 + j // 2,
                       pl.ds((j % 2) * DIM + g * 16, 16)] = v
            return carry

        lax.fori_loop(0, CH // 16, fill, 0)
        s_copy[ch] = pltpu.async_copy(
            ob, out_ref.at[pl.ds(pl.multiple_of(ow + ch * (CH // 2), 8),
                                 CH // 2)],
            ssem.at[ch % 2])

    s_copy[NCH - 2].wait()
    s_copy[NCH - 1].wait()


@jax.jit
def kernel(paths, W0, W1, W2, W3, W4, W5, W6):
    mesh = plsc.VectorSubcoreMesh(core_axis_name="c", subcore_axis_name="s")
    out = pl.kernel(
        _body,
        out_type=jax.ShapeDtypeStruct((ROWS // 2, 2 * DIM), jnp.float32),
        mesh=mesh,
        compiler_params=pltpu.CompilerParams(
            use_tc_tiling_on_sc=True, needs_layout_passes=False),
        scratch_types=[
            pltpu.VMEM((512, DIM), jnp.float32),
            pltpu.VMEM((BPC, LEVELS), jnp.int32),
            pltpu.VMEM((BPC, LEVELS), jnp.int32),
            pltpu.VMEM((CH // 2, 2 * DIM), jnp.float32),
            pltpu.VMEM((CH // 2, 2 * DIM), jnp.float32),
            pltpu.SemaphoreType.DMA,
            pltpu.SemaphoreType.DMA((2,)),
            pltpu.SemaphoreType.DMA((2,)),
        ],
    )(paths, W0, W1, W2, W3, W4, W5, W6)
    return out.reshape(BATCH, LEVELS * DIM)
